# Initial kernel scaffold; baseline (speedup 1.0000x reference)
import jax
import jax.numpy as jnp
from jax.experimental import pallas as pl


def kernel(actors, actor_idcs, actor_ctrs, nodes, node_idcs, node_ctrs, params):
    raise NotImplementedError("write your pallas kernel here")


# speedup vs baseline: 13.1895x; 13.1895x over previous
import functools

import jax
import jax.numpy as jnp
from jax import lax
from jax.experimental import pallas as pl
from jax.experimental.pallas import tpu as pltpu
from jax.experimental.pallas import tpu_sc as plsc

_N_ACTOR = 2048
_N_NODE = 16384
_D = 128
_K = 128
_CAP = 160
_DIST_TH = 0.03
_TH2_MARGIN = (0.03 * 0.03) * (1.0 + 1e-5)
_EPS = 1e-5


def _gn(x, g, b):
    m = jnp.mean(x, axis=-1, keepdims=True)
    v = jnp.mean((x - m) ** 2, axis=-1, keepdims=True)
    return (x - m) / jnp.sqrt(v + _EPS) * g + b


def _nodeproj_body(nodes_ref, wf0_ref, wf1_ref, out0_ref, out1_ref):
    n = nodes_ref[...]
    out0_ref[...] = jnp.dot(n, wf0_ref[...], preferred_element_type=jnp.float32)
    out1_ref[...] = jnp.dot(n, wf1_ref[...], preferred_element_type=jnp.float32)


def _node_projections(nodes, wf0_t, wf1_t):
    blk = 1024
    grid = _N_NODE // blk
    return pl.pallas_call(
        _nodeproj_body,
        grid=(grid,),
        in_specs=[
            pl.BlockSpec((blk, _D), lambda i: (i, 0)),
            pl.BlockSpec((_D, _D), lambda i: (0, 0)),
            pl.BlockSpec((_D, _D), lambda i: (0, 0)),
        ],
        out_specs=[
            pl.BlockSpec((blk, _D), lambda i: (i, 0)),
            pl.BlockSpec((blk, _D), lambda i: (i, 0)),
        ],
        out_shape=[
            jax.ShapeDtypeStruct((_N_NODE, _D), jnp.float32),
            jax.ShapeDtypeStruct((_N_NODE, _D), jnp.float32),
        ],
    )(nodes, wf0_t, wf1_t)


def _sc_body(acx_hbm, acy_hbm, ncx_hbm, ncy_hbm, cfw0_hbm, cfw1_hbm,
             gdx_hbm, gdy_hbm, gcf0_hbm, gcf1_hbm,
             ncx_v, ncy_v, acx_v, acy_v, nidx_v, dx_v, dy_v, gidx_v,
             rows_v, sem):
    num_cores = 2
    wid = lax.axis_index("s") * num_cores + lax.axis_index("c")
    a_per_w = _N_ACTOR // 32
    base = wid * a_per_w

    pltpu.sync_copy(ncx_hbm, ncx_v)
    pltpu.sync_copy(ncy_hbm, ncy_v)
    pltpu.sync_copy(acx_hbm, acx_v.at[pl.ds(0, _N_ACTOR)])
    pltpu.sync_copy(acy_hbm, acy_v.at[pl.ds(0, _N_ACTOR)])

    zeros_i = jnp.zeros((16,), jnp.int32)
    ones_f = jnp.ones((16,), jnp.float32)
    lane = lax.iota(jnp.int32, 16)

    def per_actor(j, carry):
        for t in range(_CAP // 16):
            nidx_v[pl.ds(t * 16, 16)] = zeros_i
            dx_v[pl.ds(t * 16, 16)] = ones_f
            dy_v[pl.ds(t * 16, 16)] = ones_f

        axv = acx_v[pl.ds(base + j, 16)][0]
        ayv = acy_v[pl.ds(base + j, 16)][0]

        def scan_chunk(t, cnt):
            ncx = ncx_v[pl.ds(t * 16, 16)]
            ncy = ncy_v[pl.ds(t * 16, 16)]
            dx = axv - ncx
            dy = ayv - ncy
            d2 = dx * dx + dy * dy
            m = d2 <= _TH2_MARGIN
            inc = plsc.all_reduce_population_count(m)[0]

            @pl.when(jnp.logical_and(inc > 0, cnt <= _CAP - 16))
            def _():
                ids = t * 16 + lane
                plsc.store_compressed(nidx_v.at[pl.ds(cnt, 16)], ids, mask=m)
                plsc.store_compressed(dx_v.at[pl.ds(cnt, 16)], dx, mask=m)
                plsc.store_compressed(dy_v.at[pl.ds(cnt, 16)], dy, mask=m)

            return cnt + inc

        lax.fori_loop(0, _N_NODE // 16, scan_chunk, jnp.int32(0))

        for t in range(_K // 16):
            gidx_v[pl.ds(t * 16, 16)] = nidx_v[pl.ds(t * 16, 16)]

        aid = base + j
        pltpu.async_copy(cfw0_hbm.at[gidx_v], rows_v, sem).wait()
        pltpu.sync_copy(rows_v, gcf0_hbm.at[pl.ds(aid * _K, _K)])
        pltpu.async_copy(cfw1_hbm.at[gidx_v], rows_v, sem).wait()
        pltpu.sync_copy(rows_v, gcf1_hbm.at[pl.ds(aid * _K, _K)])
        pltpu.sync_copy(dx_v.at[pl.ds(0, _K)], gdx_hbm.at[aid])
        pltpu.sync_copy(dy_v.at[pl.ds(0, _K)], gdy_hbm.at[aid])
        return carry

    lax.fori_loop(0, a_per_w, per_actor, jnp.int32(0))


def _sc_gather(acx, acy, ncx, ncy, cfw0, cfw1):
    mesh = plsc.VectorSubcoreMesh(core_axis_name="c", subcore_axis_name="s")
    f32 = jnp.float32
    run = pl.kernel(
        _sc_body,
        out_type=[
            jax.ShapeDtypeStruct((_N_ACTOR, _K), f32),
            jax.ShapeDtypeStruct((_N_ACTOR, _K), f32),
            jax.ShapeDtypeStruct((_N_ACTOR * _K, _D), f32),
            jax.ShapeDtypeStruct((_N_ACTOR * _K, _D), f32),
        ],
        mesh=mesh,
        compiler_params=pltpu.CompilerParams(needs_layout_passes=False),
        scratch_types=[
            pltpu.VMEM((_N_NODE,), f32),
            pltpu.VMEM((_N_NODE,), f32),
            pltpu.VMEM((_N_ACTOR + 16,), f32),
            pltpu.VMEM((_N_ACTOR + 16,), f32),
            pltpu.VMEM((_CAP,), jnp.int32),
            pltpu.VMEM((_CAP,), f32),
            pltpu.VMEM((_CAP,), f32),
            pltpu.VMEM((_K,), jnp.int32),
            pltpu.VMEM((_K, _D), f32),
            pltpu.SemaphoreType.DMA,
        ],
    )
    return run(acx, acy, ncx, ncy, cfw0, cfw1)


def _layer_body(actors_ref, gdx_ref, gdy_ref, gcf_ref,
                w1x_ref, w1y_ref, db1_ref, dw2t_ref, dg2_ref, db2_ref,
                qwt_ref, qg_ref, qb_ref,
                wdt_ref, wqt_ref, cg1_ref, cb1_ref, cw2t_ref,
                agtt_ref, ng_ref, nb_ref, lint_ref, lg_ref, lb_ref,
                out_ref, *, ba):
    f32 = jnp.float32
    res = actors_ref[...]

    q_all = jax.nn.relu(_gn(jnp.dot(res, qwt_ref[...],
                                    preferred_element_type=f32),
                            qg_ref[...], qb_ref[...]))
    qpre = jnp.dot(q_all, wqt_ref[...], preferred_element_type=f32)

    dx = gdx_ref[...]
    dy = gdy_ref[...]
    dist = jnp.sqrt(dx * dx + dy * dy)
    mask = dist <= _DIST_TH

    d1 = jax.nn.relu(dx * w1x_ref[...] + dy * w1y_ref[...] + db1_ref[...])
    d2 = jax.nn.relu(_gn(jnp.dot(d1, dw2t_ref[...],
                                 preferred_element_type=f32),
                         dg2_ref[...], db2_ref[...]))

    c = jnp.dot(d2, wdt_ref[...], preferred_element_type=f32)
    c = c + gcf_ref[...]
    c = c + jnp.broadcast_to(qpre[:, None, :], (ba, _K, _D)).reshape(ba * _K, _D)
    c = jax.nn.relu(_gn(c, cg1_ref[...], cb1_ref[...]))
    c = jnp.dot(c, cw2t_ref[...], preferred_element_type=f32)
    c = jnp.where(mask, c, 0.0)
    contrib = jnp.sum(c.reshape(ba, _K, _D), axis=1)

    a = jnp.dot(res, agtt_ref[...], preferred_element_type=f32) + contrib
    a = jax.nn.relu(_gn(a, ng_ref[...], nb_ref[...]))
    a = _gn(jnp.dot(a, lint_ref[...], preferred_element_type=f32),
            lg_ref[...], lb_ref[...])
    out_ref[...] = jax.nn.relu(a + res)


def _att_layer(actors, gdx, gdy, gcf, w):
    ba = 16
    grid = _N_ACTOR // ba
    vec = lambda: pl.BlockSpec((1, _D), lambda i: (0, 0))
    mat = lambda: pl.BlockSpec((_D, _D), lambda i: (0, 0))
    return pl.pallas_call(
        functools.partial(_layer_body, ba=ba),
        grid=(grid,),
        in_specs=[
            pl.BlockSpec((ba, _D), lambda i: (i, 0)),
            pl.BlockSpec((ba * _K, 1), lambda i: (i, 0)),
            pl.BlockSpec((ba * _K, 1), lambda i: (i, 0)),
            pl.BlockSpec((ba * _K, _D), lambda i: (i, 0)),
            vec(), vec(), vec(), mat(), vec(), vec(),
            mat(), vec(), vec(),
            mat(), mat(), vec(), vec(), mat(),
            mat(), vec(), vec(), mat(), vec(), vec(),
        ],
        out_specs=pl.BlockSpec((ba, _D), lambda i: (i, 0)),
        out_shape=jax.ShapeDtypeStruct((_N_ACTOR, _D), jnp.float32),
    )(actors, gdx, gdy, gcf, *w)


def _layer_weights(p):
    r = lambda x: x.reshape(1, _D)
    return (
        r(p['dist_w1'][:, 0]), r(p['dist_w1'][:, 1]), r(p['dist_b1']),
        p['dist_w2'].T, r(p['dist_g2']), r(p['dist_b2']),
        p['query_w'].T, r(p['query_g']), r(p['query_b']),
        p['ctx_w1'][:, :_D].T, p['ctx_w1'][:, _D:2 * _D].T,
        r(p['ctx_g1']), r(p['ctx_b1']), p['ctx_w2'].T,
        p['agt_w'].T, r(p['norm_g']), r(p['norm_b']),
        p['lin_w'].T, r(p['lin_g']), r(p['lin_b']),
    )


def kernel(actors, actor_idcs, actor_ctrs, nodes, node_idcs, node_ctrs, params):
    p0, p1 = params['att0'], params['att1']
    wf0_t = p0['ctx_w1'][:, 2 * _D:].T
    wf1_t = p1['ctx_w1'][:, 2 * _D:].T

    cfw0, cfw1 = _node_projections(nodes, wf0_t, wf1_t)

    acx = jnp.copy(actor_ctrs[:, 0])
    acy = jnp.copy(actor_ctrs[:, 1])
    ncx = jnp.copy(node_ctrs[:, 0])
    ncy = jnp.copy(node_ctrs[:, 1])

    gdx, gdy, gcf0, gcf1 = _sc_gather(acx, acy, ncx, ncy, cfw0, cfw1)
    gdx = gdx.reshape(_N_ACTOR * _K, 1)
    gdy = gdy.reshape(_N_ACTOR * _K, 1)

    a = _att_layer(actors, gdx, gdy, gcf0, _layer_weights(p0))
    a = _att_layer(a, gdx, gdy, gcf1, _layer_weights(p1))
    return a


# speedup vs baseline: 49.5220x; 3.7546x over previous
import functools

import jax
import jax.numpy as jnp
from jax import lax
from jax.experimental import pallas as pl
from jax.experimental.pallas import tpu as pltpu
from jax.experimental.pallas import tpu_sc as plsc

_N_ACTOR = 2048
_N_NODE = 16384
_D = 128
_K = 128
_CAP = 160
_CNT_MAX = _CAP - 16
_A_PER_W = _N_ACTOR // 32
_DIST_TH = 0.03
_TH2_MARGIN = (0.03 * 0.03) * (1.0 + 1e-5)
_EPS = 1e-5


def _gn(x, g, b):
    m = jnp.mean(x, axis=-1, keepdims=True)
    v = jnp.mean((x - m) ** 2, axis=-1, keepdims=True)
    return (x - m) / jnp.sqrt(v + _EPS) * g + b


def _nodeproj_body(nodes_ref, wf0_ref, wf1_ref, out_ref):
    n = nodes_ref[...]
    p0 = jnp.dot(n, wf0_ref[...], preferred_element_type=jnp.float32)
    p1 = jnp.dot(n, wf1_ref[...], preferred_element_type=jnp.float32)
    out_ref[...] = jnp.concatenate([p0, p1], axis=1)


def _node_projections(nodes, wf0_t, wf1_t):
    blk = 1024
    grid = _N_NODE // blk
    return pl.pallas_call(
        _nodeproj_body,
        grid=(grid,),
        in_specs=[
            pl.BlockSpec((blk, _D), lambda i: (i, 0)),
            pl.BlockSpec((_D, _D), lambda i: (0, 0)),
            pl.BlockSpec((_D, _D), lambda i: (0, 0)),
        ],
        out_specs=pl.BlockSpec((blk, 2 * _D), lambda i: (i, 0)),
        out_shape=jax.ShapeDtypeStruct((_N_NODE, 2 * _D), jnp.float32),
    )(nodes, wf0_t, wf1_t)


_TH2_BLOCK = (0.03 * 0.03) * (1.0 + 2e-5)
_NG = _N_NODE // 16


def _blockhit_body(acx_ref, acy_ref, ncx_ref, ncy_ref, grp_ref, out_ref):
    dx = acx_ref[...] - ncx_ref[...]
    dy = acy_ref[...] - ncy_ref[...]
    d2 = dx * dx + dy * dy
    hit = (d2 <= _TH2_BLOCK).astype(jnp.float32)
    out_ref[...] = jnp.dot(hit, grp_ref[...], preferred_element_type=jnp.float32)


def _block_hits(acx, acy, ncx, ncy, grp):
    ab, nb = 128, 2048
    return pl.pallas_call(
        _blockhit_body,
        grid=(_N_ACTOR // ab, _N_NODE // nb),
        in_specs=[
            pl.BlockSpec((ab, 1), lambda i, j: (i, 0)),
            pl.BlockSpec((ab, 1), lambda i, j: (i, 0)),
            pl.BlockSpec((1, nb), lambda i, j: (0, j)),
            pl.BlockSpec((1, nb), lambda i, j: (0, j)),
            pl.BlockSpec((nb, nb // 16), lambda i, j: (0, 0)),
        ],
        out_specs=pl.BlockSpec((ab, nb // 16), lambda i, j: (i, j)),
        out_shape=jax.ShapeDtypeStruct((_N_ACTOR, _NG), jnp.float32),
    )(acx.reshape(_N_ACTOR, 1), acy.reshape(_N_ACTOR, 1),
      ncx.reshape(1, _N_NODE), ncy.reshape(1, _N_NODE), grp)


def _sc_body(acx_hbm, acy_hbm, ncx_hbm, ncy_hbm, cfw_hbm, bh_hbm,
             gdx_hbm, gdy_hbm, gcf_hbm,
             ncx_v, ncy_v, acx_v, acy_v, nidx_v, dxc_v, dyc_v,
             nidx_a, dx_a, dy_a, cnt_a, bh0_v, bh1_v, bidx_v,
             rows0_v, rows1_v,
             sem_g, sem_o0, sem_o1, sem_b0, sem_b1):
    num_cores = 2
    wid = lax.axis_index("s") * num_cores + lax.axis_index("c")
    base = wid * _A_PER_W

    pltpu.sync_copy(ncx_hbm, ncx_v)
    pltpu.sync_copy(ncy_hbm, ncy_v)
    pltpu.sync_copy(acx_hbm.at[pl.ds(base, _A_PER_W)],
                    acx_v.at[pl.ds(0, _A_PER_W)])
    pltpu.sync_copy(acy_hbm.at[pl.ds(base, _A_PER_W)],
                    acy_v.at[pl.ds(0, _A_PER_W)])

    zeros_i = jnp.zeros((16,), jnp.int32)
    ones_f = jnp.ones((16,), jnp.float32)
    lane = lax.iota(jnp.int32, 16)

    bh = (bh0_v, bh1_v)
    bsem = (sem_b0, sem_b1)
    pltpu.async_copy(bh_hbm.at[base], bh0_v, sem_b0)

    def per_actor_pair(i, carry):
        for b in range(2):
            j = 2 * i + b
            for t in range(_CAP // 16):
                nidx_v[pl.ds(t * 16, 16)] = zeros_i
                dxc_v[pl.ds(t * 16, 16)] = ones_f
                dyc_v[pl.ds(t * 16, 16)] = ones_f

            pltpu.make_async_copy(bh_hbm.at[base], bh[b], bsem[b]).wait()
            nxt = lax.min(base + j + 1, jnp.int32(_N_ACTOR - 1))
            pltpu.async_copy(bh_hbm.at[nxt], bh[1 - b], bsem[1 - b])

            axv = acx_v[pl.ds(j, 16)][0]
            ayv = acy_v[pl.ds(j, 16)][0]

            def scan_bh(t, cbv):
                for s in range(4):
                    off = t * 64 + s * 16
                    hm = bh[b][pl.ds(off, 16)] > 0.0
                    pos = plsc.cumsum(hm.astype(jnp.int32))
                    dest = jnp.clip(cbv + pos - 1, 0, _NG - 1)
                    plsc.store_scatter(bidx_v, [dest], off + lane, mask=hm)
                    cbv = cbv + plsc.all_reduce_population_count(hm)
                return cbv

            cbv = lax.fori_loop(0, _NG // 64, scan_bh,
                                jnp.zeros((16,), jnp.int32))
            nb = cbv[0]

            def scan_hit(t, cntv):
                bid = bidx_v[pl.ds(t, 16)][0]
                off = bid * 16
                ncx = ncx_v[pl.ds(off, 16)]
                ncy = ncy_v[pl.ds(off, 16)]
                dx = axv - ncx
                dy = ayv - ncy
                d2 = dx * dx + dy * dy
                m = d2 <= _TH2_MARGIN
                pos = plsc.cumsum(m.astype(jnp.int32))
                dest = jnp.clip(cntv + pos - 1, 0, _CAP - 1)
                plsc.store_scatter(nidx_v, [dest], off + lane, mask=m)
                plsc.store_scatter(dxc_v, [dest], dx, mask=m)
                plsc.store_scatter(dyc_v, [dest], dy, mask=m)
                return cntv + plsc.all_reduce_population_count(m)

            cntv = lax.fori_loop(0, nb, scan_hit, jnp.zeros((16,), jnp.int32))
            cnt_a[pl.ds(j * 16, 16)] = cntv

            for t in range(_K // 16):
                nidx_a[j, pl.ds(t * 16, 16)] = nidx_v[pl.ds(t * 16, 16)]
                dx_a[j, pl.ds(t * 16, 16)] = dxc_v[pl.ds(t * 16, 16)]
                dy_a[j, pl.ds(t * 16, 16)] = dyc_v[pl.ds(t * 16, 16)]
        return carry

    lax.fori_loop(0, _A_PER_W // 2, per_actor_pair, jnp.int32(0))
    pltpu.make_async_copy(bh_hbm.at[base], bh0_v, sem_b0).wait()

    pltpu.sync_copy(dx_a, gdx_hbm.at[pl.ds(base, _A_PER_W)])
    pltpu.sync_copy(dy_a, gdy_hbm.at[pl.ds(base, _A_PER_W)])

    rows = (rows0_v, rows1_v)
    osem = (sem_o0, sem_o1)

    def gather_pair(i, carry):
        for b in range(2):
            j = 2 * i + b
            aid = base + j

            @pl.when(i > 0)
            def _():
                pltpu.make_async_copy(
                    rows[b], gcf_hbm.at[pl.ds(0, _K)], osem[b]).wait()

            cnt = cnt_a[pl.ds(j * 16, 16)][0]
            nblk = lax.min((cnt + 15) // 16, jnp.int32(_K // 16))
            for l in range(1, _K // 16 + 1):
                @pl.when(nblk == l)
                def _():
                    pltpu.async_copy(
                        cfw_hbm.at[nidx_a.at[j, pl.ds(0, 16 * l)]],
                        rows[b].at[pl.ds(0, 16 * l)], sem_g).wait()
            pltpu.async_copy(rows[b], gcf_hbm.at[pl.ds(aid * _K, _K)], osem[b])
        return carry

    lax.fori_loop(0, _A_PER_W // 2, gather_pair, jnp.int32(0))
    pltpu.make_async_copy(rows0_v, gcf_hbm.at[pl.ds(0, _K)], sem_o0).wait()
    pltpu.make_async_copy(rows1_v, gcf_hbm.at[pl.ds(0, _K)], sem_o1).wait()


def _sc_gather(acx, acy, ncx, ncy, cfw, bh):
    mesh = plsc.VectorSubcoreMesh(core_axis_name="c", subcore_axis_name="s")
    f32 = jnp.float32
    run = pl.kernel(
        _sc_body,
        out_type=[
            jax.ShapeDtypeStruct((_N_ACTOR, _K), f32),
            jax.ShapeDtypeStruct((_N_ACTOR, _K), f32),
            jax.ShapeDtypeStruct((_N_ACTOR * _K, 2 * _D), f32),
        ],
        mesh=mesh,
        compiler_params=pltpu.CompilerParams(needs_layout_passes=False),
        scratch_types=[
            pltpu.VMEM((_N_NODE,), f32),
            pltpu.VMEM((_N_NODE,), f32),
            pltpu.VMEM((_A_PER_W + 80,), f32),
            pltpu.VMEM((_A_PER_W + 80,), f32),
            pltpu.VMEM((_CAP,), jnp.int32),
            pltpu.VMEM((_CAP,), f32),
            pltpu.VMEM((_CAP,), f32),
            pltpu.VMEM((_A_PER_W, _K), jnp.int32),
            pltpu.VMEM((_A_PER_W, _K), f32),
            pltpu.VMEM((_A_PER_W, _K), f32),
            pltpu.VMEM((_A_PER_W * 16,), jnp.int32),
            pltpu.VMEM((_NG,), f32),
            pltpu.VMEM((_NG,), f32),
            pltpu.VMEM((_NG + 16,), jnp.int32),
            pltpu.VMEM((_K, 2 * _D), f32),
            pltpu.VMEM((_K, 2 * _D), f32),
            pltpu.SemaphoreType.DMA,
            pltpu.SemaphoreType.DMA,
            pltpu.SemaphoreType.DMA,
            pltpu.SemaphoreType.DMA,
            pltpu.SemaphoreType.DMA,
        ],
    )
    return run(acx, acy, ncx, ncy, cfw, bh)


def _layer_body(actors_ref, gdx_ref, gdy_ref, gcf_ref,
                w1x_ref, w1y_ref, db1_ref, dw2t_ref, dg2_ref, db2_ref,
                qwt_ref, qg_ref, qb_ref,
                wdt_ref, wqt_ref, cg1_ref, cb1_ref, cw2t_ref,
                agtt_ref, ng_ref, nb_ref, lint_ref, lg_ref, lb_ref,
                out_ref, *, ba):
    f32 = jnp.float32
    res = actors_ref[...]

    q_all = jax.nn.relu(_gn(jnp.dot(res, qwt_ref[...],
                                    preferred_element_type=f32),
                            qg_ref[...], qb_ref[...]))
    qpre = jnp.dot(q_all, wqt_ref[...], preferred_element_type=f32)

    dx = gdx_ref[...]
    dy = gdy_ref[...]
    dist = jnp.sqrt(dx * dx + dy * dy)
    mask = dist <= _DIST_TH

    d1 = jax.nn.relu(dx * w1x_ref[...] + dy * w1y_ref[...] + db1_ref[...])
    d2 = jax.nn.relu(_gn(jnp.dot(d1, dw2t_ref[...],
                                 preferred_element_type=f32),
                         dg2_ref[...], db2_ref[...]))

    c = jnp.dot(d2, wdt_ref[...], preferred_element_type=f32)
    c = c + gcf_ref[...]
    c = c + jnp.broadcast_to(qpre[:, None, :], (ba, _K, _D)).reshape(ba * _K, _D)
    c = jax.nn.relu(_gn(c, cg1_ref[...], cb1_ref[...]))
    c = jnp.dot(c, cw2t_ref[...], preferred_element_type=f32)
    c = jnp.where(mask, c, 0.0)
    contrib = jnp.sum(c.reshape(ba, _K, _D), axis=1)

    a = jnp.dot(res, agtt_ref[...], preferred_element_type=f32) + contrib
    a = jax.nn.relu(_gn(a, ng_ref[...], nb_ref[...]))
    a = _gn(jnp.dot(a, lint_ref[...], preferred_element_type=f32),
            lg_ref[...], lb_ref[...])
    out_ref[...] = jax.nn.relu(a + res)


def _att_layer(actors, gdx, gdy, gcf, w, lcol):
    ba = 32
    grid = _N_ACTOR // ba
    vec = lambda: pl.BlockSpec((1, _D), lambda i: (0, 0))
    mat = lambda: pl.BlockSpec((_D, _D), lambda i: (0, 0))
    return pl.pallas_call(
        functools.partial(_layer_body, ba=ba),
        grid=(grid,),
        in_specs=[
            pl.BlockSpec((ba, _D), lambda i: (i, 0)),
            pl.BlockSpec((ba * _K, 1), lambda i: (i, 0)),
            pl.BlockSpec((ba * _K, 1), lambda i: (i, 0)),
            pl.BlockSpec((ba * _K, _D), lambda i, c=lcol: (i, c)),
            vec(), vec(), vec(), mat(), vec(), vec(),
            mat(), vec(), vec(),
            mat(), mat(), vec(), vec(), mat(),
            mat(), vec(), vec(), mat(), vec(), vec(),
        ],
        out_specs=pl.BlockSpec((ba, _D), lambda i: (i, 0)),
        out_shape=jax.ShapeDtypeStruct((_N_ACTOR, _D), jnp.float32),
    )(actors, gdx, gdy, gcf, *w)


def _layer_weights(p):
    r = lambda x: x.reshape(1, _D)
    return (
        r(p['dist_w1'][:, 0]), r(p['dist_w1'][:, 1]), r(p['dist_b1']),
        p['dist_w2'].T, r(p['dist_g2']), r(p['dist_b2']),
        p['query_w'].T, r(p['query_g']), r(p['query_b']),
        p['ctx_w1'][:, :_D].T, p['ctx_w1'][:, _D:2 * _D].T,
        r(p['ctx_g1']), r(p['ctx_b1']), p['ctx_w2'].T,
        p['agt_w'].T, r(p['norm_g']), r(p['norm_b']),
        p['lin_w'].T, r(p['lin_g']), r(p['lin_b']),
    )


def kernel(actors, actor_idcs, actor_ctrs, nodes, node_idcs, node_ctrs, params):
    p0, p1 = params['att0'], params['att1']
    wf0_t = p0['ctx_w1'][:, 2 * _D:].T
    wf1_t = p1['ctx_w1'][:, 2 * _D:].T

    cfw = _node_projections(nodes, wf0_t, wf1_t)

    acx = jnp.copy(actor_ctrs[:, 0])
    acy = jnp.copy(actor_ctrs[:, 1])
    ncx = jnp.copy(node_ctrs[:, 0])
    ncy = jnp.copy(node_ctrs[:, 1])

    grp = jnp.kron(jnp.eye(_D, dtype=jnp.float32),
                   jnp.ones((16, 1), jnp.float32))
    bh = _block_hits(acx, acy, ncx, ncy, grp)

    gdx, gdy, gcf = _sc_gather(acx, acy, ncx, ncy, cfw, bh)
    gdx = gdx.reshape(_N_ACTOR * _K, 1)
    gdy = gdy.reshape(_N_ACTOR * _K, 1)

    a = _att_layer(actors, gdx, gdy, gcf, _layer_weights(p0), 0)
    a = _att_layer(a, gdx, gdy, gcf, _layer_weights(p1), 1)
    return a
